# Initial kernel scaffold; baseline (speedup 1.0000x reference)
import jax
import jax.numpy as jnp
from jax.experimental import pallas as pl


def kernel(x, edge_index, batch_index, W1, b1, W_ih, W_hh, b_ih, b_hh, Wp1, bp1, Wp2, bp2):
    raise NotImplementedError("write your pallas kernel here")


# speedup vs baseline: 6.5376x; 6.5376x over previous
import functools

import jax
import jax.numpy as jnp
from jax import lax
from jax.experimental import pallas as pl
from jax.experimental.pallas import tpu as pltpu
from jax.experimental.pallas import tpu_sc as plsc

_N = 10000
_E = 320000
_D = 128
_B = 256
_NUM_CONVS = 3
_STEPS = 3

_NC = 2
_NS = 16
_NW = _NC * _NS
_EPT = _E // _NW
_K = 80
_NCH = _EPT // _K
_NPAD = 10240
_RPT = _NPAD // _NS


def _sc_agg_body(hx, srcs, dsts, zeros, out, srcv, dstv, rows, sem, agg):
    c = lax.axis_index("c")
    s = lax.axis_index("s")
    wid = s * _NC + c
    pltpu.sync_copy(srcs.at[wid], srcv)
    pltpu.sync_copy(dsts.at[wid], dstv)
    pltpu.sync_copy(zeros, agg.at[pl.ds(s * _RPT, _RPT)])
    plsc.subcore_barrier()

    def body(j, carry):
        pltpu.async_copy(hx.at[srcv.at[j]], rows, sem).wait()
        pltpu.sync_copy(rows, agg.at[dstv.at[j]], add=True)
        return carry

    lax.fori_loop(0, _NCH, body, 0)
    plsc.subcore_barrier()
    pltpu.sync_copy(agg.at[pl.ds(s * _RPT, _RPT)],
                    out.at[c, pl.ds(s * _RPT, _RPT)])


def _sc_agg(hx, src_r, dst_r, zeros):
    mesh = plsc.VectorSubcoreMesh(
        core_axis_name="c", subcore_axis_name="s",
        num_cores=_NC, num_subcores=_NS)
    f = pl.kernel(
        _sc_agg_body,
        out_type=jax.ShapeDtypeStruct((_NC, _NPAD, _D), jnp.float32),
        mesh=mesh,
        scratch_types=[
            pltpu.VMEM((_NCH, _K), jnp.int32),
            pltpu.VMEM((_NCH, _K), jnp.int32),
            pltpu.VMEM((_K, _D), jnp.float32),
            pltpu.SemaphoreType.DMA,
            pltpu.VMEM_SHARED((_NPAD, _D), jnp.float32),
        ],
    )
    return f(hx, src_r, dst_r, zeros)


def _conv_body(hx_ref, agg_ref, w_ref, b_ref, o_ref):
    h = hx_ref[...] + agg_ref[0, :_N] + agg_ref[1, :_N]
    y = jnp.dot(h, w_ref[...], preferred_element_type=jnp.float32)
    o_ref[...] = jnp.maximum(y + b_ref[...], 0.0)


def _conv(hx, agg, w1t, b1row):
    return pl.pallas_call(
        _conv_body,
        out_shape=jax.ShapeDtypeStruct((_N, _D), jnp.float32),
    )(hx, agg, w1t, b1row)


def _s2s_body(hx_ref, bidx_ref, wih_ref, whh_ref, bias_ref,
              wp1_ref, bp1_ref, wp2_ref, bp2_ref, o_ref):
    hx = hx_ref[...]
    bidx = bidx_ref[...]
    cols = lax.broadcasted_iota(jnp.int32, (_N, _B), 1)
    mask = bidx == cols
    wih = wih_ref[...]
    whh = whh_ref[...]
    bias = bias_ref[...]
    q_star = jnp.zeros((_B, 2 * _D), jnp.float32)
    h = jnp.zeros((_B, _D), jnp.float32)
    c = jnp.zeros((_B, _D), jnp.float32)
    for _ in range(_STEPS):
        gates = (jnp.dot(q_star, wih, preferred_element_type=jnp.float32)
                 + jnp.dot(h, whh, preferred_element_type=jnp.float32)
                 + bias)
        i_g = gates[:, :_D]
        f_g = gates[:, _D:2 * _D]
        g_g = gates[:, 2 * _D:3 * _D]
        o_g = gates[:, 3 * _D:]
        c = jax.nn.sigmoid(f_g) * c + jax.nn.sigmoid(i_g) * jnp.tanh(g_g)
        h = jax.nn.sigmoid(o_g) * jnp.tanh(c)
        e = lax.dot_general(hx, h, (((1,), (1,)), ((), ())),
                            preferred_element_type=jnp.float32,
                 precision=lax.Precision.HIGHEST)
        em = jnp.where(mask, e, -jnp.inf)
        emax = jnp.max(em, axis=0)
        emax = jnp.where(jnp.isfinite(emax), emax, 0.0)
        a = jnp.exp(em - emax[None, :])
        denom = jnp.sum(a, axis=0)
        a = a / (denom[None, :] + 1e-16)
        r = lax.dot_general(a, hx, (((0,), (0,)), ((), ())),
                            preferred_element_type=jnp.float32,
                 precision=lax.Precision.HIGHEST)
        q_star = jnp.concatenate([h, r], axis=1)
    h1 = jnp.maximum(
        jnp.dot(q_star, wp1_ref[...], preferred_element_type=jnp.float32)
        + bp1_ref[...], 0.0)
    o_ref[...] = (jnp.dot(h1, wp2_ref[...], preferred_element_type=jnp.float32)
                  + bp2_ref[...])


def _s2s(hx, bidx, wih_t, whh_t, bias, wp1_t, bp1row, wp2_t, bp2row):
    return pl.pallas_call(
        _s2s_body,
        out_shape=jax.ShapeDtypeStruct((_B, _D), jnp.float32),
    )(hx, bidx, wih_t, whh_t, bias, wp1_t, bp1row, wp2_t, bp2row)


def kernel(x, edge_index, batch_index, W1, b1, W_ih, W_hh, b_ih, b_hh,
           Wp1, bp1, Wp2, bp2):
    hx = jnp.pad(x, ((0, 0), (0, _D - x.shape[1])))
    src_r = edge_index[0].reshape(_NW, _NCH, _K)
    dst_r = edge_index[1].reshape(_NW, _NCH, _K)
    zeros = jnp.zeros((_RPT, _D), jnp.float32)
    w1t = W1.T
    b1row = b1.reshape(1, _D)
    for _ in range(_NUM_CONVS):
        agg = _sc_agg(hx, src_r, dst_r, zeros)
        hx = _conv(hx, agg, w1t, b1row)
    bias = (b_ih + b_hh).reshape(1, 4 * _D)
    out = _s2s(hx, batch_index.reshape(_N, 1), W_ih.T, W_hh.T, bias,
               Wp1.T, bp1.reshape(1, _D), Wp2.T, bp2.reshape(1, _D))
    return out


# speedup vs baseline: 7.4582x; 1.1408x over previous
import jax
import jax.numpy as jnp
from jax import lax
from jax.experimental import pallas as pl
from jax.experimental.pallas import tpu as pltpu
from jax.experimental.pallas import tpu_sc as plsc

_N = 10000
_E = 320000
_D = 128
_DH = _D // 2
_B = 256
_NUM_CONVS = 3
_STEPS = 3

_NC = 2
_NS = 16
_EPT = _E // _NS
_K = 80
_NCH = _EPT // _K
_NPAD = 10240
_RPT = _NPAD // _NS


def _sc_agg_body(hx2, srcs, dsts, zeros, out, srcv, dstv, rows, sem, agg):
    c = lax.axis_index("c")
    s = lax.axis_index("s")
    pltpu.sync_copy(srcs.at[s], srcv)
    pltpu.sync_copy(dsts.at[s], dstv)
    pltpu.sync_copy(zeros, agg.at[pl.ds(s * _RPT, _RPT)])
    plsc.subcore_barrier()

    hxc = hx2.at[c]
    def body(i, carry):
        @pl.when(i < _NCH)
        def _():
            pltpu.async_copy(
                hxc.at[srcv.at[i]], rows.at[pl.ds((i % 2) * _K, _K)], sem)

        @pl.when(i > 0)
        def _():
            j = i - 1
            p = (j % 2) * _K
            pltpu.make_async_copy(
                hxc.at[srcv.at[j]], rows.at[pl.ds(p, _K)], sem).wait()
            pltpu.sync_copy(rows.at[pl.ds(p, _K)], agg.at[dstv.at[j]],
                            add=True)
        return carry

    lax.fori_loop(0, _NCH + 1, body, 0)
    plsc.subcore_barrier()
    pltpu.sync_copy(agg.at[pl.ds(s * _RPT, _RPT)],
                    out.at[c, pl.ds(s * _RPT, _RPT)])


def _sc_agg(hx2, src_r, dst_r, zeros):
    mesh = plsc.VectorSubcoreMesh(
        core_axis_name="c", subcore_axis_name="s",
        num_cores=_NC, num_subcores=_NS)
    f = pl.kernel(
        _sc_agg_body,
        out_type=jax.ShapeDtypeStruct((_NC, _NPAD, _DH), jnp.float32),
        mesh=mesh,
        compiler_params=pltpu.CompilerParams(use_tc_tiling_on_sc=False),
        scratch_types=[
            pltpu.VMEM((_NCH, _K), jnp.int32),
            pltpu.VMEM((_NCH, _K), jnp.int32),
            pltpu.VMEM((2 * _K, _DH), jnp.float32),
            pltpu.SemaphoreType.DMA,
            pltpu.VMEM_SHARED((_NPAD, _DH), jnp.float32),
        ],
    )
    return f(hx2, src_r, dst_r, zeros)


def _conv_body(hx2_ref, agg_ref, w_ref, b_ref, o_ref):
    h = jnp.concatenate(
        [hx2_ref[0] + agg_ref[0, :_N], hx2_ref[1] + agg_ref[1, :_N]], axis=1)
    y = jnp.dot(h, w_ref[...], preferred_element_type=jnp.float32)
    y = jnp.maximum(y + b_ref[...], 0.0)
    o_ref[0] = y[:, :_DH]
    o_ref[1] = y[:, _DH:]


def _conv(hx2, agg, w1t, b1row):
    return pl.pallas_call(
        _conv_body,
        out_shape=jax.ShapeDtypeStruct((_NC, _N, _DH), jnp.float32),
    )(hx2, agg, w1t, b1row)


def _s2s_body(hx_ref, bidx_ref, wih_ref, whh_ref, bias_ref,
              wp1_ref, bp1_ref, wp2_ref, bp2_ref, o_ref):
    hx = hx_ref[...]
    bidx = bidx_ref[...]
    cols = lax.broadcasted_iota(jnp.int32, (_N, _B), 1)
    mask = bidx == cols
    wih = wih_ref[...]
    whh = whh_ref[...]
    bias = bias_ref[...]
    q_star = jnp.zeros((_B, 2 * _D), jnp.float32)
    h = jnp.zeros((_B, _D), jnp.float32)
    c = jnp.zeros((_B, _D), jnp.float32)
    for _ in range(_STEPS):
        gates = (jnp.dot(q_star, wih, preferred_element_type=jnp.float32)
                 + jnp.dot(h, whh, preferred_element_type=jnp.float32)
                 + bias)
        i_g = gates[:, :_D]
        f_g = gates[:, _D:2 * _D]
        g_g = gates[:, 2 * _D:3 * _D]
        o_g = gates[:, 3 * _D:]
        c = jax.nn.sigmoid(f_g) * c + jax.nn.sigmoid(i_g) * jnp.tanh(g_g)
        h = jax.nn.sigmoid(o_g) * jnp.tanh(c)
        e = lax.dot_general(hx, h, (((1,), (1,)), ((), ())),
                            preferred_element_type=jnp.float32,
                            precision=lax.Precision.HIGHEST)
        em = jnp.where(mask, e, -jnp.inf)
        emax = jnp.max(em, axis=0)
        emax = jnp.where(jnp.isfinite(emax), emax, 0.0)
        a = jnp.exp(em - emax[None, :])
        denom = jnp.sum(a, axis=0)
        a = a / (denom[None, :] + 1e-16)
        r = lax.dot_general(a, hx, (((0,), (0,)), ((), ())),
                            preferred_element_type=jnp.float32,
                            precision=lax.Precision.HIGHEST)
        q_star = jnp.concatenate([h, r], axis=1)
    h1 = jnp.maximum(
        jnp.dot(q_star, wp1_ref[...], preferred_element_type=jnp.float32)
        + bp1_ref[...], 0.0)
    o_ref[...] = (jnp.dot(h1, wp2_ref[...], preferred_element_type=jnp.float32)
                  + bp2_ref[...])


def _s2s(hx, bidx, wih_t, whh_t, bias, wp1_t, bp1row, wp2_t, bp2row):
    return pl.pallas_call(
        _s2s_body,
        out_shape=jax.ShapeDtypeStruct((_B, _D), jnp.float32),
    )(hx, bidx, wih_t, whh_t, bias, wp1_t, bp1row, wp2_t, bp2row)


def kernel(x, edge_index, batch_index, W1, b1, W_ih, W_hh, b_ih, b_hh,
           Wp1, bp1, Wp2, bp2):
    hx2 = jnp.stack([x, jnp.zeros((_N, _DH), jnp.float32)])
    src_r = edge_index[0].reshape(_NS, _NCH, _K)
    dst_r = edge_index[1].reshape(_NS, _NCH, _K)
    zeros = jnp.zeros((_RPT, _DH), jnp.float32)
    w1t = W1.T
    b1row = b1.reshape(1, _D)
    for _ in range(_NUM_CONVS):
        agg = _sc_agg(hx2, src_r, dst_r, zeros)
        hx2 = _conv(hx2, agg, w1t, b1row)
    bias = (b_ih + b_hh).reshape(1, 4 * _D)
    hx = jnp.concatenate([hx2[0], hx2[1]], axis=1)
    out = _s2s(hx, batch_index.reshape(_N, 1), W_ih.T, W_hh.T, bias,
               Wp1.T, bp1.reshape(1, _D), Wp2.T, bp2.reshape(1, _D))
    return out


# speedup vs baseline: 9.9120x; 1.3290x over previous
import jax
import jax.numpy as jnp
from jax import lax
from jax.experimental import pallas as pl
from jax.experimental.pallas import tpu as pltpu
from jax.experimental.pallas import tpu_sc as plsc

_N = 10000
_E = 320000
_D = 128
_DH = _D // 2
_B = 256
_NUM_CONVS = 3
_STEPS = 3

_NC = 2
_NS = 16
_EPT = _E // _NS
_K = 80
_NCH = _EPT // _K
_NPAD = 10240
_RPT = _NPAD // _NS


def _sc_agg_body(hx2, srcs, dsts, zeros, out, srcv, dstv, rows, sem, agg):
    c = lax.axis_index("c")
    s = lax.axis_index("s")
    pltpu.sync_copy(srcs.at[s], srcv)
    pltpu.sync_copy(dsts.at[s], dstv)
    pltpu.sync_copy(zeros, agg.at[pl.ds(s * _RPT, _RPT)])
    plsc.subcore_barrier()

    hxc = hx2.at[c]
    def body(i, carry):
        @pl.when(i < _NCH)
        def _():
            pltpu.async_copy(
                hxc.at[srcv.at[i]], rows.at[pl.ds((i % 4) * _K, _K)], sem)

        @pl.when(i >= 3)
        def _():
            j = i - 3
            p = (j % 4) * _K
            pltpu.make_async_copy(
                hxc.at[srcv.at[j]], rows.at[pl.ds(p, _K)], sem).wait()
            pltpu.sync_copy(rows.at[pl.ds(p, _K)], agg.at[dstv.at[j]],
                            add=True)
        return carry

    lax.fori_loop(0, _NCH + 3, body, 0)
    plsc.subcore_barrier()
    pltpu.sync_copy(agg.at[pl.ds(s * _RPT, _RPT)],
                    out.at[c, pl.ds(s * _RPT, _RPT)])


def _sc_agg(hx2, src_r, dst_r, zeros):
    mesh = plsc.VectorSubcoreMesh(
        core_axis_name="c", subcore_axis_name="s",
        num_cores=_NC, num_subcores=_NS)
    f = pl.kernel(
        _sc_agg_body,
        out_type=jax.ShapeDtypeStruct((_NC, _NPAD, _DH), jnp.float32),
        mesh=mesh,
        compiler_params=pltpu.CompilerParams(use_tc_tiling_on_sc=False),
        scratch_types=[
            pltpu.VMEM((_NCH, _K), jnp.int32),
            pltpu.VMEM((_NCH, _K), jnp.int32),
            pltpu.VMEM((4 * _K, _DH), jnp.float32),
            pltpu.SemaphoreType.DMA,
            pltpu.VMEM_SHARED((_NPAD, _DH), jnp.float32),
        ],
    )
    return f(hx2, src_r, dst_r, zeros)


def _conv_body(hx2_ref, agg_ref, w_ref, b_ref, o_ref):
    h = jnp.concatenate(
        [hx2_ref[0] + agg_ref[0, :_N], hx2_ref[1] + agg_ref[1, :_N]], axis=1)
    y = jnp.dot(h, w_ref[...], preferred_element_type=jnp.float32)
    y = jnp.maximum(y + b_ref[...], 0.0)
    o_ref[0] = y[:, :_DH]
    o_ref[1] = y[:, _DH:]


def _conv(hx2, agg, w1t, b1row):
    return pl.pallas_call(
        _conv_body,
        out_shape=jax.ShapeDtypeStruct((_NC, _N, _DH), jnp.float32),
    )(hx2, agg, w1t, b1row)


def _s2s_body(hx_ref, bidx_ref, wih_ref, whh_ref, bias_ref,
              wp1_ref, bp1_ref, wp2_ref, bp2_ref, o_ref):
    hx = hx_ref[...]
    bidx = bidx_ref[...]
    cols = lax.broadcasted_iota(jnp.int32, (_N, _B), 1)
    mask = bidx == cols
    wih = wih_ref[...]
    whh = whh_ref[...]
    bias = bias_ref[...]
    q_star = jnp.zeros((_B, 2 * _D), jnp.float32)
    h = jnp.zeros((_B, _D), jnp.float32)
    c = jnp.zeros((_B, _D), jnp.float32)
    for _ in range(_STEPS):
        gates = (jnp.dot(q_star, wih, preferred_element_type=jnp.float32)
                 + jnp.dot(h, whh, preferred_element_type=jnp.float32)
                 + bias)
        i_g = gates[:, :_D]
        f_g = gates[:, _D:2 * _D]
        g_g = gates[:, 2 * _D:3 * _D]
        o_g = gates[:, 3 * _D:]
        c = jax.nn.sigmoid(f_g) * c + jax.nn.sigmoid(i_g) * jnp.tanh(g_g)
        h = jax.nn.sigmoid(o_g) * jnp.tanh(c)
        e = lax.dot_general(hx, h, (((1,), (1,)), ((), ())),
                            preferred_element_type=jnp.float32,
                            precision=lax.Precision.HIGHEST)
        em = jnp.where(mask, e, -jnp.inf)
        emax = jnp.max(em, axis=0)
        emax = jnp.where(jnp.isfinite(emax), emax, 0.0)
        a = jnp.exp(em - emax[None, :])
        denom = jnp.sum(a, axis=0)
        a = a / (denom[None, :] + 1e-16)
        r = lax.dot_general(a, hx, (((0,), (0,)), ((), ())),
                            preferred_element_type=jnp.float32,
                            precision=lax.Precision.HIGHEST)
        q_star = jnp.concatenate([h, r], axis=1)
    h1 = jnp.maximum(
        jnp.dot(q_star, wp1_ref[...], preferred_element_type=jnp.float32)
        + bp1_ref[...], 0.0)
    o_ref[...] = (jnp.dot(h1, wp2_ref[...], preferred_element_type=jnp.float32)
                  + bp2_ref[...])


def _s2s(hx, bidx, wih_t, whh_t, bias, wp1_t, bp1row, wp2_t, bp2row):
    return pl.pallas_call(
        _s2s_body,
        out_shape=jax.ShapeDtypeStruct((_B, _D), jnp.float32),
    )(hx, bidx, wih_t, whh_t, bias, wp1_t, bp1row, wp2_t, bp2row)


def kernel(x, edge_index, batch_index, W1, b1, W_ih, W_hh, b_ih, b_hh,
           Wp1, bp1, Wp2, bp2):
    hx2 = jnp.stack([x, jnp.zeros((_N, _DH), jnp.float32)])
    src_r = edge_index[0].reshape(_NS, _NCH, _K)
    dst_r = edge_index[1].reshape(_NS, _NCH, _K)
    zeros = jnp.zeros((_RPT, _DH), jnp.float32)
    w1t = W1.T
    b1row = b1.reshape(1, _D)
    for _ in range(_NUM_CONVS):
        agg = _sc_agg(hx2, src_r, dst_r, zeros)
        hx2 = _conv(hx2, agg, w1t, b1row)
    bias = (b_ih + b_hh).reshape(1, 4 * _D)
    hx = jnp.concatenate([hx2[0], hx2[1]], axis=1)
    out = _s2s(hx, batch_index.reshape(_N, 1), W_ih.T, W_hh.T, bias,
               Wp1.T, bp1.reshape(1, _D), Wp2.T, bp2.reshape(1, _D))
    return out


# speedup vs baseline: 10.0199x; 1.0109x over previous
import jax
import jax.numpy as jnp
from jax import lax
from jax.experimental import pallas as pl
from jax.experimental.pallas import tpu as pltpu
from jax.experimental.pallas import tpu_sc as plsc

_N = 10000
_E = 320000
_D = 128
_DH = _D // 2
_B = 256
_NUM_CONVS = 3
_STEPS = 3

_NC = 2
_NS = 16
_EPT = _E // _NS
_K = 80
_NCH = _EPT // _K
_NPAD = 10240
_RPT = _NPAD // _NS


def _sc_agg_body(hx2, srcs, dsts, zeros, out, srcv, dstv, rows, sem, agg):
    c = lax.axis_index("c")
    s = lax.axis_index("s")
    pltpu.sync_copy(srcs.at[s], srcv)
    pltpu.sync_copy(dsts.at[s], dstv)
    pltpu.sync_copy(zeros, agg.at[pl.ds(s * _RPT, _RPT)])
    plsc.subcore_barrier()

    hxc = hx2.at[c]
    def body(u, carry):
        for d in range(2):
            i = 2 * u + d

            @pl.when(i < _NCH)
            def _(i=i):
                pltpu.async_copy(
                    hxc.at[srcv.at[i]],
                    rows.at[pl.ds((i % 8) * _K, _K)], sem)

            @pl.when((i >= 7) & (i < _NCH + 7))
            def _(i=i):
                j = i - 7
                p = (j % 8) * _K
                pltpu.make_async_copy(
                    hxc.at[srcv.at[j]], rows.at[pl.ds(p, _K)], sem).wait()
                pltpu.sync_copy(rows.at[pl.ds(p, _K)], agg.at[dstv.at[j]],
                                add=True)
        return carry

    lax.fori_loop(0, (_NCH + 8) // 2, body, 0)
    plsc.subcore_barrier()
    pltpu.sync_copy(agg.at[pl.ds(s * _RPT, _RPT)],
                    out.at[c, pl.ds(s * _RPT, _RPT)])


def _sc_agg(hx2, src_r, dst_r, zeros):
    mesh = plsc.VectorSubcoreMesh(
        core_axis_name="c", subcore_axis_name="s",
        num_cores=_NC, num_subcores=_NS)
    f = pl.kernel(
        _sc_agg_body,
        out_type=jax.ShapeDtypeStruct((_NC, _NPAD, _DH), jnp.float32),
        mesh=mesh,
        compiler_params=pltpu.CompilerParams(use_tc_tiling_on_sc=False),
        scratch_types=[
            pltpu.VMEM((_NCH, _K), jnp.int32),
            pltpu.VMEM((_NCH, _K), jnp.int32),
            pltpu.VMEM((8 * _K, _DH), jnp.float32),
            pltpu.SemaphoreType.DMA,
            pltpu.VMEM_SHARED((_NPAD, _DH), jnp.float32),
        ],
    )
    return f(hx2, src_r, dst_r, zeros)


def _conv_body(hx2_ref, agg_ref, w_ref, b_ref, o_ref):
    h = jnp.concatenate(
        [hx2_ref[0] + agg_ref[0, :_N], hx2_ref[1] + agg_ref[1, :_N]], axis=1)
    y = jnp.dot(h, w_ref[...], preferred_element_type=jnp.float32)
    y = jnp.maximum(y + b_ref[...], 0.0)
    o_ref[0] = y[:, :_DH]
    o_ref[1] = y[:, _DH:]


def _conv(hx2, agg, w1t, b1row):
    return pl.pallas_call(
        _conv_body,
        out_shape=jax.ShapeDtypeStruct((_NC, _N, _DH), jnp.float32),
    )(hx2, agg, w1t, b1row)


def _s2s_body(hx_ref, bidx_ref, wih_ref, whh_ref, bias_ref,
              wp1_ref, bp1_ref, wp2_ref, bp2_ref, o_ref):
    hx = hx_ref[...]
    bidx = bidx_ref[...]
    cols = lax.broadcasted_iota(jnp.int32, (_N, _B), 1)
    mask = bidx == cols
    wih = wih_ref[...]
    whh = whh_ref[...]
    bias = bias_ref[...]
    q_star = jnp.zeros((_B, 2 * _D), jnp.float32)
    h = jnp.zeros((_B, _D), jnp.float32)
    c = jnp.zeros((_B, _D), jnp.float32)
    for _ in range(_STEPS):
        gates = (jnp.dot(q_star, wih, preferred_element_type=jnp.float32)
                 + jnp.dot(h, whh, preferred_element_type=jnp.float32)
                 + bias)
        i_g = gates[:, :_D]
        f_g = gates[:, _D:2 * _D]
        g_g = gates[:, 2 * _D:3 * _D]
        o_g = gates[:, 3 * _D:]
        c = jax.nn.sigmoid(f_g) * c + jax.nn.sigmoid(i_g) * jnp.tanh(g_g)
        h = jax.nn.sigmoid(o_g) * jnp.tanh(c)
        e = lax.dot_general(hx, h, (((1,), (1,)), ((), ())),
                            preferred_element_type=jnp.float32,
                            precision=lax.Precision.HIGHEST)
        em = jnp.where(mask, e, -jnp.inf)
        emax = jnp.max(em, axis=0)
        emax = jnp.where(jnp.isfinite(emax), emax, 0.0)
        a = jnp.exp(em - emax[None, :])
        denom = jnp.sum(a, axis=0)
        a = a / (denom[None, :] + 1e-16)
        r = lax.dot_general(a, hx, (((0,), (0,)), ((), ())),
                            preferred_element_type=jnp.float32,
                            precision=lax.Precision.HIGHEST)
        q_star = jnp.concatenate([h, r], axis=1)
    h1 = jnp.maximum(
        jnp.dot(q_star, wp1_ref[...], preferred_element_type=jnp.float32)
        + bp1_ref[...], 0.0)
    o_ref[...] = (jnp.dot(h1, wp2_ref[...], preferred_element_type=jnp.float32)
                  + bp2_ref[...])


def _s2s(hx, bidx, wih_t, whh_t, bias, wp1_t, bp1row, wp2_t, bp2row):
    return pl.pallas_call(
        _s2s_body,
        out_shape=jax.ShapeDtypeStruct((_B, _D), jnp.float32),
    )(hx, bidx, wih_t, whh_t, bias, wp1_t, bp1row, wp2_t, bp2row)


def kernel(x, edge_index, batch_index, W1, b1, W_ih, W_hh, b_ih, b_hh,
           Wp1, bp1, Wp2, bp2):
    hx2 = jnp.stack([x, jnp.zeros((_N, _DH), jnp.float32)])
    src_r = edge_index[0].reshape(_NS, _NCH, _K)
    dst_r = edge_index[1].reshape(_NS, _NCH, _K)
    zeros = jnp.zeros((_RPT, _DH), jnp.float32)
    w1t = W1.T
    b1row = b1.reshape(1, _D)
    for _ in range(_NUM_CONVS):
        agg = _sc_agg(hx2, src_r, dst_r, zeros)
        hx2 = _conv(hx2, agg, w1t, b1row)
    bias = (b_ih + b_hh).reshape(1, 4 * _D)
    hx = jnp.concatenate([hx2[0], hx2[1]], axis=1)
    out = _s2s(hx, batch_index.reshape(_N, 1), W_ih.T, W_hh.T, bias,
               Wp1.T, bp1.reshape(1, _D), Wp2.T, bp2.reshape(1, _D))
    return out
